# baseline (speedup 1.0000x reference)
import functools

import jax
import jax.numpy as jnp
from jax import lax
from jax.experimental import pallas as pl
from jax.experimental.pallas import tpu as pltpu
from jax.experimental.pallas import tpu_sc as plsc

_NEG = -1e37


def _sc_gather_rows(Wv, sid3, lab3, S, B):
    D = Wv.shape[1]
    info = plsc.get_sparse_core_info()
    NC, NS = info.num_cores, info.num_subcores
    NW = NC * NS
    s_per = S // NW
    l_per = B // NW
    CH = 128
    s_ch = s_per // CH
    l_ch = l_per // CH

    mesh = plsc.VectorSubcoreMesh(core_axis_name="c", subcore_axis_name="s")

    @functools.partial(
        pl.kernel,
        mesh=mesh,
        out_type=[
            jax.ShapeDtypeStruct((S, D), jnp.float32),
            jax.ShapeDtypeStruct((B, D), jnp.float32),
        ],
        scratch_types=[
            pltpu.VMEM((s_ch, CH), jnp.int32),
            pltpu.VMEM((s_per, D), jnp.float32),
            pltpu.VMEM((l_ch, CH), jnp.int32),
            pltpu.VMEM((l_per, D), jnp.float32),
            pltpu.SemaphoreType.DMA,
        ],
    )
    def gather_k(w_hbm, sid_hbm, lab_hbm,
                 sw_out, tw_out,
                 sidx_v, srows_v, lidx_v, lrows_v, sem):
        wid = lax.axis_index("s") * NC + lax.axis_index("c")
        sbase = wid * s_per
        lbase = wid * l_per
        pltpu.sync_copy(sid_hbm.at[wid], sidx_v)
        pltpu.sync_copy(lab_hbm.at[wid], lidx_v)
        handles = []
        for j in range(s_ch):
            handles.append(pltpu.async_copy(
                w_hbm.at[sidx_v.at[j]],
                srows_v.at[pl.ds(j * CH, CH), :], sem))
        for j in range(l_ch):
            handles.append(pltpu.async_copy(
                w_hbm.at[lidx_v.at[j]],
                lrows_v.at[pl.ds(j * CH, CH), :], sem))
        for h in handles:
            h.wait()
        pltpu.sync_copy(srows_v, sw_out.at[pl.ds(sbase, s_per)])
        pltpu.sync_copy(lrows_v, tw_out.at[pl.ds(lbase, l_per)])

    return gather_k(Wv, sid3, lab3)


def _sc_gather_bias(b, sid3, lab3, S, B):
    info = plsc.get_sparse_core_info()
    NC, NS = info.num_cores, info.num_subcores
    NW = NC * NS
    s_per = S // NW
    l_per = B // NW
    CH = 128
    s_ch = s_per // CH
    l_ch = l_per // CH

    mesh = plsc.VectorSubcoreMesh(core_axis_name="c", subcore_axis_name="s")

    @functools.partial(
        pl.kernel,
        mesh=mesh,
        compiler_params=pltpu.CompilerParams(use_tc_tiling_on_sc=False),
        out_type=[
            jax.ShapeDtypeStruct((S,), jnp.float32),
            jax.ShapeDtypeStruct((B,), jnp.float32),
        ],
        scratch_types=[
            pltpu.VMEM((s_ch, CH), jnp.int32),
            pltpu.VMEM((s_per,), jnp.float32),
            pltpu.VMEM((l_ch, CH), jnp.int32),
            pltpu.VMEM((l_per,), jnp.float32),
            pltpu.SemaphoreType.DMA,
        ],
    )
    def gather_k(b_hbm, sid_hbm, lab_hbm,
                 sb_out, tb_out,
                 sidx_v, sb_v, lidx_v, lb_v, sem):
        wid = lax.axis_index("s") * NC + lax.axis_index("c")
        sbase = wid * s_per
        lbase = wid * l_per
        pltpu.sync_copy(sid_hbm.at[wid], sidx_v)
        pltpu.sync_copy(lab_hbm.at[wid], lidx_v)
        handles = []
        for j in range(s_ch):
            handles.append(pltpu.async_copy(
                b_hbm.at[sidx_v.at[j]],
                sb_v.at[pl.ds(j * CH, CH)], sem))
        for j in range(l_ch):
            handles.append(pltpu.async_copy(
                b_hbm.at[lidx_v.at[j]],
                lb_v.at[pl.ds(j * CH, CH)], sem))
        for h in handles:
            h.wait()
        pltpu.sync_copy(sb_v, sb_out.at[pl.ds(sbase, s_per)])
        pltpu.sync_copy(lb_v, tb_out.at[pl.ds(lbase, l_per)])

    return gather_k(b, sid3, lab3)


def _prologue_body(gs_ref, grs_ref, gt_ref, grt_ref, swk_ref, tw_ref):
    def extract(g, grp):
        out = jnp.zeros((g.shape[0], 32), jnp.float32)
        for k in range(4):
            out = jnp.where(grp == k, g[:, 32 * k:32 * (k + 1)], out)
        return out

    sw = extract(gs_ref[...], grs_ref[...])
    swk_ref[...] = jnp.concatenate(
        [jnp.zeros((1, 32), jnp.float32), sw], axis=0)
    tw_ref[...] = extract(gt_ref[...], grt_ref[...])


def _tc_prologue(gs, grs, gt, grt):
    S = gs.shape[0]
    B = gt.shape[0]
    return pl.pallas_call(
        _prologue_body,
        out_shape=[
            jax.ShapeDtypeStruct((S + 1, 32), jnp.float32),
            jax.ShapeDtypeStruct((B, 32), jnp.float32),
        ],
    )(gs, grs, gt, grt)


def _tc_body(x_ref, wk_ref, ids_ref, bias_ref, sfreq_ref,
             lab_ref, tw_ref, tb_ref, tf_ref, out_ref):
    x = x_ref[...]
    s = lax.dot_general(x, wk_ref[...], (((1,), (1,)), ((), ())),
                        preferred_element_type=jnp.float32)
    row = bias_ref[...] - jnp.log(sfreq_ref[...])
    s = s + row
    hit = lab_ref[...] == ids_ref[...]
    s = jnp.where(hit, jnp.float32(_NEG), s)
    t = (jnp.sum(x * tw_ref[...], axis=1, keepdims=True)
         + tb_ref[...] - jnp.log(tf_ref[...]))
    out_ref[...] = s
    out_ref[:, 0:1] = t


def _tc_logits(x, wk, ids_p, bias_p, sfreq_p, lab2, tw, tb2, tf2):
    BATCH, D = x.shape
    W1 = wk.shape[0]
    BR = 256
    nb = BATCH // BR
    return pl.pallas_call(
        _tc_body,
        grid=(nb,),
        in_specs=[
            pl.BlockSpec((BR, D), lambda i: (i, 0)),
            pl.BlockSpec((W1, D), lambda i: (0, 0)),
            pl.BlockSpec((1, W1), lambda i: (0, 0)),
            pl.BlockSpec((1, W1), lambda i: (0, 0)),
            pl.BlockSpec((1, W1), lambda i: (0, 0)),
            pl.BlockSpec((BR, 1), lambda i: (i, 0)),
            pl.BlockSpec((BR, D), lambda i: (i, 0)),
            pl.BlockSpec((BR, 1), lambda i: (i, 0)),
            pl.BlockSpec((BR, 1), lambda i: (i, 0)),
        ],
        out_specs=pl.BlockSpec((BR, W1), lambda i: (i, 0)),
        out_shape=jax.ShapeDtypeStruct((BATCH, W1), jnp.float32),
        compiler_params=pltpu.CompilerParams(
            dimension_semantics=("arbitrary",)),
    )(x, wk, ids_p, bias_p, sfreq_p, lab2, tw, tb2, tf2)


def kernel(inputs, W, b, true_freq, sample_freq, labels, sample_ids):
    S = sample_ids.shape[0]
    B = labels.shape[0]
    NW, CH = 32, 128

    Wv = W.reshape(-1, 128)
    sid3 = (sample_ids >> 2).reshape(NW, -1, CH)
    lab3 = (labels >> 2).reshape(NW, -1, CH)
    sidb3 = sample_ids.reshape(NW, -1, CH)
    labb3 = labels.reshape(NW, -1, CH)

    gs, gt = _sc_gather_rows(Wv, sid3, lab3, S, B)
    sb, tb = _sc_gather_bias(b, sidb3, labb3, S, B)

    swk, tw = _tc_prologue(gs, (sample_ids & 3)[:, None],
                           gt, (labels & 3)[:, None])

    ids_p = jnp.concatenate(
        [jnp.full((1,), -1, jnp.int32), sample_ids])[None, :]
    bias_p = jnp.concatenate(
        [jnp.zeros((1,), jnp.float32), sb])[None, :]
    sfreq_p = jnp.concatenate(
        [jnp.ones((1,), jnp.float32), sample_freq])[None, :]

    return _tc_logits(inputs, swk, ids_p, bias_p, sfreq_p,
                      labels[:, None], tw, tb[:, None], true_freq[:, None])


# speedup vs baseline: 1.3854x; 1.3854x over previous
import functools

import jax
import jax.numpy as jnp
from jax import lax
from jax.experimental import pallas as pl
from jax.experimental.pallas import tpu as pltpu
from jax.experimental.pallas import tpu_sc as plsc

_NEG = -1e37


def _sc_gather_bias(b, sid3, lab3, S, B):
    info = plsc.get_sparse_core_info()
    NC, NS = info.num_cores, info.num_subcores
    NW = NC * NS
    s_per = S // NW
    l_per = B // NW
    CH = 128
    s_ch = s_per // CH
    l_ch = l_per // CH

    mesh = plsc.VectorSubcoreMesh(core_axis_name="c", subcore_axis_name="s")

    @functools.partial(
        pl.kernel,
        mesh=mesh,
        compiler_params=pltpu.CompilerParams(use_tc_tiling_on_sc=False),
        out_type=[
            jax.ShapeDtypeStruct((S,), jnp.float32),
            jax.ShapeDtypeStruct((B,), jnp.float32),
        ],
        scratch_types=[
            pltpu.VMEM((s_ch, CH), jnp.int32),
            pltpu.VMEM((s_per,), jnp.float32),
            pltpu.VMEM((l_ch, CH), jnp.int32),
            pltpu.VMEM((l_per,), jnp.float32),
            pltpu.SemaphoreType.DMA,
        ],
    )
    def gather_k(b_hbm, sid_hbm, lab_hbm,
                 sb_out, tb_out,
                 sidx_v, sb_v, lidx_v, lb_v, sem):
        wid = lax.axis_index("s") * NC + lax.axis_index("c")
        pltpu.sync_copy(sid_hbm.at[wid], sidx_v)
        pltpu.sync_copy(lab_hbm.at[wid], lidx_v)
        handles = []
        for j in range(s_ch):
            handles.append(pltpu.async_copy(
                b_hbm.at[sidx_v.at[j]],
                sb_v.at[pl.ds(j * CH, CH)], sem))
        for j in range(l_ch):
            handles.append(pltpu.async_copy(
                b_hbm.at[lidx_v.at[j]],
                lb_v.at[pl.ds(j * CH, CH)], sem))
        for h in handles:
            h.wait()
        pltpu.sync_copy(sb_v, sb_out.at[pl.ds(wid * s_per, s_per)])
        pltpu.sync_copy(lb_v, tb_out.at[pl.ds(wid * l_per, l_per)])

    return gather_k(b, sid3, lab3)


def _gather_prologue_body(sid_ref, lab_ref, w_hbm, sb_ref, sf_ref,
                          tb_ref, tf_ref, wk_ref, c_ref, tw_ref,
                          rows_s, rows_t, sem):
    S = rows_s.shape[0]
    B = rows_t.shape[0]

    def fire_s(i, _):
        pltpu.make_async_copy(
            w_hbm.at[pl.ds(sid_ref[i], 1), :], rows_s.at[pl.ds(i, 1), :],
            sem).start()
        return 0

    def fire_l(i, _):
        pltpu.make_async_copy(
            w_hbm.at[pl.ds(lab_ref[i], 1), :], rows_t.at[pl.ds(i, 1), :],
            sem).start()
        return 0

    lax.fori_loop(0, S, fire_s, 0, unroll=8)
    lax.fori_loop(0, B, fire_l, 0, unroll=8)
    pltpu.make_async_copy(w_hbm.at[pl.ds(0, S), :], rows_s, sem).wait()
    pltpu.make_async_copy(w_hbm.at[pl.ds(0, B), :], rows_t, sem).wait()

    aug = jnp.concatenate(
        [rows_s[...], sb_ref[...], -jnp.log(sf_ref[...])], axis=1)
    wk_ref[...] = jnp.concatenate(
        [jnp.zeros((1, aug.shape[1]), jnp.float32), aug], axis=0)
    c_ref[...] = tb_ref[...] - jnp.log(tf_ref[...])
    tw_ref[...] = rows_t[...]


def _tc_gather_prologue(sample_ids, labels, W, sb2, sf2, tb2, tf2):
    S = sample_ids.shape[0]
    B = labels.shape[0]
    D = W.shape[1]
    return pl.pallas_call(
        _gather_prologue_body,
        in_specs=[
            pl.BlockSpec(memory_space=pltpu.SMEM),
            pl.BlockSpec(memory_space=pltpu.SMEM),
            pl.BlockSpec(memory_space=pl.ANY),
            pl.BlockSpec((S, 1), lambda: (0, 0)),
            pl.BlockSpec((S, 1), lambda: (0, 0)),
            pl.BlockSpec((B, 1), lambda: (0, 0)),
            pl.BlockSpec((B, 1), lambda: (0, 0)),
        ],
        out_shape=[
            jax.ShapeDtypeStruct((S + 1, 34), jnp.float32),
            jax.ShapeDtypeStruct((B, 1), jnp.float32),
            jax.ShapeDtypeStruct((B, D), jnp.float32),
        ],
        scratch_shapes=[
            pltpu.VMEM((S, D), jnp.float32),
            pltpu.VMEM((B, D), jnp.float32),
            pltpu.SemaphoreType.DMA,
        ],
    )(sample_ids, labels, W, sb2, sf2, tb2, tf2)


def _tc_body(x_ref, wk_ref, ids_ref, lab_ref, tw_ref, c_ref, out_ref):
    x = x_ref[...]
    xa = jnp.concatenate(
        [x, jnp.ones((x.shape[0], 2), jnp.float32)], axis=1)
    s = lax.dot_general(xa, wk_ref[...], (((1,), (1,)), ((), ())),
                        preferred_element_type=jnp.float32)
    hit = lab_ref[...] == ids_ref[...]
    s = jnp.where(hit, jnp.float32(_NEG), s)
    t = jnp.sum(x * tw_ref[...], axis=1, keepdims=True) + c_ref[...]
    out_ref[...] = s
    out_ref[:, 0:1] = t


def _tc_logits(x, wk, ids_p, lab2, tw, c2):
    BATCH, D = x.shape
    W1 = wk.shape[0]
    KA = wk.shape[1]
    BR = 256
    nb = BATCH // BR
    return pl.pallas_call(
        _tc_body,
        grid=(nb,),
        in_specs=[
            pl.BlockSpec((BR, D), lambda i: (i, 0)),
            pl.BlockSpec((W1, KA), lambda i: (0, 0)),
            pl.BlockSpec((1, W1), lambda i: (0, 0)),
            pl.BlockSpec((BR, 1), lambda i: (i, 0)),
            pl.BlockSpec((BR, D), lambda i: (i, 0)),
            pl.BlockSpec((BR, 1), lambda i: (i, 0)),
        ],
        out_specs=pl.BlockSpec((BR, W1), lambda i: (i, 0)),
        out_shape=jax.ShapeDtypeStruct((BATCH, W1), jnp.float32),
        compiler_params=pltpu.CompilerParams(
            dimension_semantics=("arbitrary",)),
    )(x, wk, ids_p, lab2, tw, c2)


def kernel(inputs, W, b, true_freq, sample_freq, labels, sample_ids):
    S = sample_ids.shape[0]
    B = labels.shape[0]
    NW, CH = 32, 128

    sid3 = sample_ids.reshape(NW, -1, CH)
    lab3 = labels.reshape(NW, -1, CH)

    sb, tb = _sc_gather_bias(b, sid3, lab3, S, B)

    wk, c2, tw = _tc_gather_prologue(
        sample_ids, labels, W, sb[:, None], sample_freq[:, None],
        tb[:, None], true_freq[:, None])

    ids_p = jnp.concatenate(
        [jnp.full((1,), -1, jnp.int32), sample_ids])[None, :]

    return _tc_logits(inputs, wk, ids_p, labels[:, None], tw, c2)


# speedup vs baseline: 1.4021x; 1.0121x over previous
import functools

import jax
import jax.numpy as jnp
from jax import lax
from jax.experimental import pallas as pl
from jax.experimental.pallas import tpu as pltpu
from jax.experimental.pallas import tpu_sc as plsc

_NEG = -1e37
_NSEM = 8


def _sc_gather_bias(b, sid3, lab3, S, B):
    info = plsc.get_sparse_core_info()
    NC, NS = info.num_cores, info.num_subcores
    NW = NC * NS
    s_per = S // NW
    l_per = B // NW
    CH = 128
    s_ch = s_per // CH
    l_ch = l_per // CH

    mesh = plsc.VectorSubcoreMesh(core_axis_name="c", subcore_axis_name="s")

    @functools.partial(
        pl.kernel,
        mesh=mesh,
        compiler_params=pltpu.CompilerParams(use_tc_tiling_on_sc=False),
        out_type=[
            jax.ShapeDtypeStruct((S,), jnp.float32),
            jax.ShapeDtypeStruct((B,), jnp.float32),
        ],
        scratch_types=[
            pltpu.VMEM((s_ch, CH), jnp.int32),
            pltpu.VMEM((s_per,), jnp.float32),
            pltpu.VMEM((l_ch, CH), jnp.int32),
            pltpu.VMEM((l_per,), jnp.float32),
            pltpu.SemaphoreType.DMA,
        ],
    )
    def gather_k(b_hbm, sid_hbm, lab_hbm,
                 sb_out, tb_out,
                 sidx_v, sb_v, lidx_v, lb_v, sem):
        wid = lax.axis_index("s") * NC + lax.axis_index("c")
        pltpu.sync_copy(sid_hbm.at[wid], sidx_v)
        pltpu.sync_copy(lab_hbm.at[wid], lidx_v)
        handles = []
        for j in range(s_ch):
            handles.append(pltpu.async_copy(
                b_hbm.at[sidx_v.at[j]],
                sb_v.at[pl.ds(j * CH, CH)], sem))
        for j in range(l_ch):
            handles.append(pltpu.async_copy(
                b_hbm.at[lidx_v.at[j]],
                lb_v.at[pl.ds(j * CH, CH)], sem))
        for h in handles:
            h.wait()
        pltpu.sync_copy(sb_v, sb_out.at[pl.ds(wid * s_per, s_per)])
        pltpu.sync_copy(lb_v, tb_out.at[pl.ds(wid * l_per, l_per)])

    return gather_k(b, sid3, lab3)


def _gather_prologue_body(sid_ref, lab_ref, w_hbm, sfp_ref, tf_ref,
                          wk_ref, lf_ref, c_ref, tw_ref, sems):
    S = wk_ref.shape[0] - 1
    B = tw_ref.shape[0]

    def fire_s(i, _):
        for j in range(_NSEM):
            k = i * _NSEM + j
            pltpu.make_async_copy(
                w_hbm.at[pl.ds(sid_ref[k], 1), :],
                wk_ref.at[pl.ds(k + 1, 1), :], sems.at[j]).start()
        return 0

    def fire_l(i, _):
        for j in range(_NSEM):
            k = i * _NSEM + j
            pltpu.make_async_copy(
                w_hbm.at[pl.ds(lab_ref[k], 1), :],
                tw_ref.at[pl.ds(k, 1), :], sems.at[j]).start()
        return 0

    lax.fori_loop(0, S // _NSEM, fire_s, 0)
    lax.fori_loop(0, B // _NSEM, fire_l, 0)
    wk_ref[0:1, :] = jnp.zeros((1, 32), jnp.float32)
    lf_ref[...] = -jnp.log(sfp_ref[...])
    c_ref[...] = -jnp.log(tf_ref[...])
    nper = (S + B) // _NSEM
    for j in range(_NSEM):
        pltpu.make_async_copy(
            w_hbm.at[pl.ds(0, nper), :],
            wk_ref.at[pl.ds(1, nper), :], sems.at[j]).wait()


def _tc_gather_prologue(sample_ids, labels, W, sfreq_p, tf2):
    S = sample_ids.shape[0]
    B = labels.shape[0]
    D = W.shape[1]
    W1 = S + 1
    return pl.pallas_call(
        _gather_prologue_body,
        in_specs=[
            pl.BlockSpec(memory_space=pltpu.SMEM),
            pl.BlockSpec(memory_space=pltpu.SMEM),
            pl.BlockSpec(memory_space=pl.ANY),
            pl.BlockSpec((1, W1), lambda: (0, 0)),
            pl.BlockSpec((B, 1), lambda: (0, 0)),
        ],
        out_shape=[
            jax.ShapeDtypeStruct((W1, D), jnp.float32),
            jax.ShapeDtypeStruct((1, W1), jnp.float32),
            jax.ShapeDtypeStruct((B, 1), jnp.float32),
            jax.ShapeDtypeStruct((B, D), jnp.float32),
        ],
        scratch_shapes=[
            pltpu.SemaphoreType.DMA((_NSEM,)),
        ],
    )(sample_ids, labels, W, sfreq_p, tf2)


def _tc_body(x_ref, wk_ref, ids_ref, sb_ref, lf_ref, lab_ref,
             tw_ref, tb_ref, c_ref, out_ref):
    x = x_ref[...]
    s = lax.dot_general(x, wk_ref[...], (((1,), (1,)), ((), ())),
                        preferred_element_type=jnp.float32)
    s = s + (sb_ref[...] + lf_ref[...])
    hit = lab_ref[...] == ids_ref[...]
    s = jnp.where(hit, jnp.float32(_NEG), s)
    t = (jnp.sum(x * tw_ref[...], axis=1, keepdims=True)
         + tb_ref[...] + c_ref[...])
    out_ref[...] = s
    out_ref[:, 0:1] = t


def _tc_logits(x, wk, ids_p, sb_p, lf, lab2, tw, tb2, c2):
    BATCH, D = x.shape
    W1 = wk.shape[0]
    BR = 256
    nb = BATCH // BR
    return pl.pallas_call(
        _tc_body,
        grid=(nb,),
        in_specs=[
            pl.BlockSpec((BR, D), lambda i: (i, 0)),
            pl.BlockSpec((W1, D), lambda i: (0, 0)),
            pl.BlockSpec((1, W1), lambda i: (0, 0)),
            pl.BlockSpec((1, W1), lambda i: (0, 0)),
            pl.BlockSpec((1, W1), lambda i: (0, 0)),
            pl.BlockSpec((BR, 1), lambda i: (i, 0)),
            pl.BlockSpec((BR, D), lambda i: (i, 0)),
            pl.BlockSpec((BR, 1), lambda i: (i, 0)),
            pl.BlockSpec((BR, 1), lambda i: (i, 0)),
        ],
        out_specs=pl.BlockSpec((BR, W1), lambda i: (i, 0)),
        out_shape=jax.ShapeDtypeStruct((BATCH, W1), jnp.float32),
        compiler_params=pltpu.CompilerParams(
            dimension_semantics=("arbitrary",)),
    )(x, wk, ids_p, sb_p, lf, lab2, tw, tb2, c2)


def kernel(inputs, W, b, true_freq, sample_freq, labels, sample_ids):
    S = sample_ids.shape[0]
    B = labels.shape[0]
    NW, CH = 32, 128

    sid3 = sample_ids.reshape(NW, -1, CH)
    lab3 = labels.reshape(NW, -1, CH)

    sb, tb = _sc_gather_bias(b, sid3, lab3, S, B)

    sfreq_p = jnp.concatenate(
        [jnp.ones((1,), jnp.float32), sample_freq])[None, :]
    wk, lf, c2, tw = _tc_gather_prologue(
        sample_ids, labels, W, sfreq_p, true_freq[:, None])

    ids_p = jnp.concatenate(
        [jnp.full((1,), -1, jnp.int32), sample_ids])[None, :]
    sb_p = jnp.concatenate(
        [jnp.zeros((1,), jnp.float32), sb])[None, :]

    return _tc_logits(inputs, wk, ids_p, sb_p, lf,
                      labels[:, None], tw, tb[:, None], c2)
